# Initial kernel scaffold; baseline (speedup 1.0000x reference)
import jax
import jax.numpy as jnp
from jax.experimental import pallas as pl


def kernel(embedding, W_e1, b_e1, W_e2, b_e2, W_h1, b_h1, W_h2, b_h2, W_he, b_he, W_q, W_k, W_v, b_v, W_s1, b_s1, W_s2, b_s2, W_d1, b_d1, W_d2, b_d2):
    raise NotImplementedError("write your pallas kernel here")


# speedup vs baseline: 40.8484x; 40.8484x over previous
import jax
import jax.numpy as jnp
from jax.experimental import pallas as pl

_B, _N, _D = 2048, 16, 64
_BB = 16
_GRID = _B // _BB


def _lrelu(x):
    return jnp.where(x >= 0, x, 0.01 * x)


def _pad8(w):
    return jnp.pad(w, ((0, 8 - w.shape[0]), (0, 0)))


def _body(flat_ref, We1_ref, be1_ref, We2_ref, be2_ref, Wh1h_ref, Wh1t2_ref,
          Wh1s_ref, bh1_ref, whard_ref, bhard_ref, Wq_ref, Wkt_ref, Wks_ref,
          Wvt_ref, Wvs_ref, bv_ref, Ws1q_ref, Ws1k_ref, bs1_ref, ws2_ref,
          bs2_ref, Wd1h_ref, Wd1o_ref, bd1_ref, Wd2_ref, bd2_ref, out_ref):
    f32 = jnp.float32
    dot = lambda a, b: jnp.dot(a, b, preferred_element_type=f32)
    flat = flat_ref[...]

    h = _lrelu(dot(flat, We1_ref[...]) + be1_ref[...])
    h = _lrelu(dot(h, We2_ref[...]) + be2_ref[...])

    tpart = dot(h, Wh1h_ref[...]) - dot(flat, Wh1t2_ref[...]) + bh1_ref[...]
    spart = dot(flat, Wh1s_ref[...])
    kt = dot(flat, Wkt_ref[...])
    ks = dot(flat, Wks_ref[...])
    vt = dot(flat, Wvt_ref[...])
    vs = dot(flat, Wvs_ref[...])
    qs1 = dot(dot(h, Wq_ref[...]), Ws1q_ref[...]) + bs1_ref[...]

    def n3(x):
        return x.reshape(_BB, _N, x.shape[-1])

    hh = jnp.maximum(n3(tpart)[:, :, None, :] + n3(spart)[:, None, :, :], 0.0)
    hard_logit = jnp.sum(hh * whard_ref[...].reshape(1, 1, 1, _D), axis=-1)
    hard = jax.nn.sigmoid(hard_logit + bhard_ref[0, 0])

    k4 = _lrelu(n3(kt)[:, :, None, :] + n3(ks)[:, None, :, :])
    sk = dot(k4.reshape(_BB * _N * _N, _D), Ws1k_ref[...])
    spre = jnp.maximum(sk.reshape(_BB, _N, _N, _D)
                       + n3(qs1)[:, :, None, :], 0.0)
    scores = jnp.sum(spre * ws2_ref[...].reshape(1, 1, 1, _D), axis=-1)
    scores = scores + bs2_ref[0, 0]

    ti = jax.lax.broadcasted_iota(jnp.int32, (_N, _N), 0)
    si = jax.lax.broadcasted_iota(jnp.int32, (_N, _N), 1)
    scores = jnp.where((ti == si)[None, :, :], -1e30, scores)
    m = jnp.max(scores, axis=2, keepdims=True)
    ex = jnp.exp(scores - m)
    w = ex / jnp.sum(ex, axis=2, keepdims=True) * hard

    v4 = _lrelu(n3(vt)[:, :, None, :] + n3(vs)[:, None, :, :]
                + bv_ref[...].reshape(1, 1, 1, _D))
    agg = jnp.sum(v4 * w[:, :, :, None], axis=2)

    dec = _lrelu(dot(h, Wd1h_ref[...])
                 + dot(agg.reshape(_BB * _N, _D), Wd1o_ref[...])
                 + bd1_ref[...])
    dec = _lrelu(dot(dec, Wd2_ref[...]) + bd2_ref[...])
    out_ref[...] = dec


def kernel(embedding, W_e1, b_e1, W_e2, b_e2, W_h1, b_h1, W_h2, b_h2,
           W_he, b_he, W_q, W_k, W_v, b_v, W_s1, b_s1, W_s2, b_s2,
           W_d1, b_d1, W_d2, b_d2):
    flatp = jnp.pad(embedding.reshape(_B * _N, 5), ((0, 0), (0, 3)))

    whe_diff = W_he[:, 1] - W_he[:, 0]
    w_hard = (W_h2 @ whe_diff).reshape(1, _D)
    b_hard = (b_h2 @ whe_diff + b_he[1] - b_he[0]).reshape(1, 1)
    weights = (
        _pad8(W_e1), b_e1.reshape(1, -1), W_e2, b_e2.reshape(1, -1),
        W_h1[:_D], _pad8(W_h1[_D + 5:]),
        _pad8(W_h1[_D:_D + 5]) + _pad8(W_h1[_D + 5:]), b_h1.reshape(1, -1),
        w_hard, b_hard, W_q,
        _pad8(W_k[:5]), _pad8(W_k[5:]),
        _pad8(W_v[:5]), _pad8(W_v[5:]), b_v.reshape(1, -1),
        W_s1[:_D], W_s1[_D:], b_s1.reshape(1, -1),
        W_s2.reshape(1, _D), b_s2.reshape(1, 1),
        W_d1[:_D], W_d1[_D:], b_d1.reshape(1, -1),
        W_d2, b_d2.reshape(1, -1),
    )
    in_specs = [pl.BlockSpec((_BB * _N, 8), lambda i: (i, 0))] + [
        pl.BlockSpec(w.shape, lambda i: (0, 0)) for w in weights
    ]
    return pl.pallas_call(
        _body,
        grid=(_GRID,),
        in_specs=in_specs,
        out_specs=pl.BlockSpec((_BB * _N, 2 * _D), lambda i: (i, 0)),
        out_shape=jax.ShapeDtypeStruct((_B * _N, 2 * _D), jnp.float32),
    )(flatp, *weights)
